# baseline (speedup 1.0000x reference)
import functools

import jax
import jax.numpy as jnp
from jax import lax
from jax.experimental import pallas as pl
from jax.experimental.pallas import tpu as pltpu
from jax.experimental.pallas import tpu_sc as plsc

_NC = 2
_NS = 16
_NW = _NC * _NS

_N = 4096
_S = 2048
_ND = 4
_NCONT = 32
_DIM = 512
_L = 16
_NROW = _ND + _NCONT
_HEAD = 8
_TAIL = _NROW - _HEAD
_PW = _N // _NW
_PP = 2
_STEPS = _PW // _PP


def _sc_body(idx_hbm, csp_hbm, dtab_hbm, ctab_hbm, out_hbm,
             idx_v, ctab_v, sbuf, gbuf, abuf, cbuf, gsem, ssem, osem0, osem1):
    wid = lax.axis_index("s") * _NC + lax.axis_index("c")
    p0 = wid * _PW
    bsel = p0 // _S
    sbase = p0 % _S
    osem = (osem0, osem1)

    pltpu.sync_copy(idx_hbm.at[pl.ds(p0 * _ND, _PW * _ND)], idx_v)
    pltpu.sync_copy(ctab_hbm, ctab_v)

    def prefetch(s, nb):
        pltpu.async_copy(csp_hbm.at[pl.ds(p0 + s * _PP, _PP)],
                         sbuf.at[nb], ssem)

    def do_step(s, nb):
        pltpu.make_async_copy(csp_hbm.at[pl.ds(0, _PP)],
                              sbuf.at[nb], ssem).wait()

        off = pl.multiple_of(s * (_PP * _ND), 8)
        gh = pltpu.async_copy(dtab_hbm.at[idx_v.at[pl.ds(off, _PP * _ND)]],
                              gbuf.at[nb], gsem)

        def cj(jr, c):
            j = jr + (_HEAD - _ND)
            sp = [sbuf[nb, pp, j, pl.ds(0, _L)] for pp in range(_PP)]
            for k in range(_DIM // _L):
                ks = pl.ds(k * _L, _L)
                row = ctab_v[j, ks]
                for pp in range(_PP):
                    cbuf[nb, pp, jr, ks] = sp[pp] * row
            return c
        lax.fori_loop(0, _TAIL, cj, 0)

        sph = [[sbuf[nb, pp, j, pl.ds(0, _L)] for j in range(_HEAD - _ND)]
               for pp in range(_PP)]

        gh.wait()

        def ck(k, c):
            ks = pl.ds(k * _L, _L)
            for j in range(_HEAD - _ND):
                row = ctab_v[j, ks]
                for pp in range(_PP):
                    abuf[nb, pp, _ND + j, ks] = sph[pp][j] * row
            for pp in range(_PP):
                for r in range(_ND):
                    abuf[nb, pp, r, ks] = gbuf[nb, pp * _ND + r, ks]
            return c
        lax.fori_loop(0, _DIM // _L, ck, 0)

        spos = sbase + s * _PP
        for pp in range(_PP):
            pltpu.async_copy(abuf.at[nb, pp],
                             out_hbm.at[bsel, spos + pp, pl.ds(0, _HEAD)],
                             osem[nb])
            pltpu.async_copy(cbuf.at[nb, pp],
                             out_hbm.at[bsel, spos + pp, pl.ds(_HEAD, _TAIL)],
                             osem[nb])

        prefetch(jnp.minimum(s + 2, _STEPS - 1), nb)

    def drain_out(nb):
        pltpu.make_async_copy(out_hbm.at[0, pl.ds(0, _PP), pl.ds(0, _HEAD)],
                              abuf.at[nb], osem[nb]).wait()
        pltpu.make_async_copy(out_hbm.at[0, pl.ds(0, _PP), pl.ds(_HEAD, _TAIL)],
                              cbuf.at[nb], osem[nb]).wait()

    def drain_splat(nb):
        pltpu.make_async_copy(csp_hbm.at[pl.ds(0, _PP)],
                              sbuf.at[nb], ssem).wait()

    prefetch(0, 0)
    prefetch(1, 1)
    do_step(0, 0)
    do_step(1, 1)

    def outer(s2, c):
        for nb in range(2):
            drain_out(nb)
            do_step(s2 * 2 + nb, nb)
        return c
    lax.fori_loop(1, _STEPS // 2, outer, 0)
    drain_out(0)
    drain_out(1)
    drain_splat(0)
    drain_splat(1)


@jax.jit
def _sc_call(flat_idx, cont_splat, disc_table, cont_table):
    mesh = plsc.VectorSubcoreMesh(core_axis_name="c", subcore_axis_name="s")
    f = functools.partial(
        pl.kernel, _sc_body, mesh=mesh,
        out_type=jax.ShapeDtypeStruct((_N // _S, _S, _NROW, _DIM), jnp.float32),
        scratch_types=[
            pltpu.VMEM((_PW * _ND,), jnp.int32),
            pltpu.VMEM((_NCONT, _DIM), jnp.float32),
            pltpu.VMEM((2, _PP, _NCONT, _L), jnp.float32),
            pltpu.VMEM((2, _PP * _ND, _DIM), jnp.float32),
            pltpu.VMEM((2, _PP, _HEAD, _DIM), jnp.float32),
            pltpu.VMEM((2, _PP, _TAIL, _DIM), jnp.float32),
            pltpu.SemaphoreType.DMA,
            pltpu.SemaphoreType.DMA,
            pltpu.SemaphoreType.DMA,
            pltpu.SemaphoreType.DMA,
        ],
    )()
    return f(flat_idx, cont_splat, disc_table, cont_table)


def kernel(discrete_actions, continuous_actions, disc_table, cont_table, offsets):
    b, s, n_disc = discrete_actions.shape
    n_cont = continuous_actions.shape[-1]
    dim = disc_table.shape[-1]
    n = b * s
    flat_idx = (discrete_actions + offsets[None, None, :]).reshape(n * n_disc)
    cont_splat = jnp.broadcast_to(
        continuous_actions.reshape(n, n_cont)[:, :, None], (n, n_cont, _L))
    out = _sc_call(flat_idx, cont_splat, disc_table, cont_table)
    return out.reshape(b, s, n_disc + n_cont, dim)


# speedup vs baseline: 1.3506x; 1.3506x over previous
import functools

import jax
import jax.numpy as jnp
from jax import lax
from jax.experimental import pallas as pl
from jax.experimental.pallas import tpu as pltpu
from jax.experimental.pallas import tpu_sc as plsc

_NC = 2
_NS = 16
_NW = _NC * _NS

_N = 4096
_S = 2048
_ND = 4
_NCONT = 32
_DIM = 512
_L = 16
_NROW = _ND + _NCONT
_HEAD = 8
_TAIL = _NROW - _HEAD
_PW = _N // _NW
_PP = 2
_STEPS = _PW // _PP


def _sc_body(idx_hbm, csp_hbm, dtab_hbm, ctab_hbm, out_hbm,
             idx_v, ctab_v, sbuf, gbuf, abuf, cbuf, gsem, ssem, osem0, osem1):
    wid = lax.axis_index("s") * _NC + lax.axis_index("c")
    p0 = wid * _PW
    bsel = p0 // _S
    sbase = p0 % _S
    osem = (osem0, osem1)

    pltpu.sync_copy(idx_hbm.at[pl.ds(p0 * _ND, _PW * _ND)], idx_v)
    pltpu.sync_copy(ctab_hbm, ctab_v)

    def prefetch(s, nb):
        pltpu.async_copy(csp_hbm.at[pl.ds(p0 + s * _PP, _PP)],
                         sbuf.at[nb], ssem)

    def do_step(s, nb):
        pltpu.make_async_copy(csp_hbm.at[pl.ds(0, _PP)],
                              sbuf.at[nb], ssem).wait()

        off = pl.multiple_of(s * (_PP * _ND), 8)
        gh = pltpu.async_copy(dtab_hbm.at[idx_v.at[pl.ds(off, _PP * _ND)]],
                              gbuf.at[nb], gsem)

        G = 4
        NG = _DIM // _L // G

        def cj(jr, c):
            j = jr + (_HEAD - _ND)
            sp = [sbuf[nb, pp, j, pl.ds(0, _L)] for pp in range(_PP)]
            rows = [ctab_v[j, pl.ds(t * _L, _L)] for t in range(G)]
            for g in range(NG):
                base = g * G
                prods = [[sp[pp] * rows[t] for pp in range(_PP)]
                         for t in range(G)]
                if g + 1 < NG:
                    rows = [ctab_v[j, pl.ds((base + G + t) * _L, _L)]
                            for t in range(G)]
                for t in range(G):
                    for pp in range(_PP):
                        cbuf[nb, pp, jr, pl.ds((base + t) * _L, _L)] = \
                            prods[t][pp]
            return c
        lax.fori_loop(0, _TAIL, cj, 0)

        sph = [[sbuf[nb, pp, j, pl.ds(0, _L)] for j in range(_HEAD - _ND)]
               for pp in range(_PP)]

        gh.wait()

        def ck(kg, c):
            for t in range(G):
                ks = pl.ds((kg * G + t) * _L, _L)
                rows = [ctab_v[j, ks] for j in range(_HEAD - _ND)]
                gcp = [[gbuf[nb, pp * _ND + r, ks] for r in range(_ND)]
                       for pp in range(_PP)]
                for j in range(_HEAD - _ND):
                    for pp in range(_PP):
                        abuf[nb, pp, _ND + j, ks] = sph[pp][j] * rows[j]
                for pp in range(_PP):
                    for r in range(_ND):
                        abuf[nb, pp, r, ks] = gcp[pp][r]
            return c
        lax.fori_loop(0, NG, ck, 0)

        spos = sbase + s * _PP
        for pp in range(_PP):
            pltpu.async_copy(abuf.at[nb, pp],
                             out_hbm.at[bsel, spos + pp, pl.ds(0, _HEAD)],
                             osem[nb])
            pltpu.async_copy(cbuf.at[nb, pp],
                             out_hbm.at[bsel, spos + pp, pl.ds(_HEAD, _TAIL)],
                             osem[nb])

        prefetch(jnp.minimum(s + 2, _STEPS - 1), nb)

    def drain_out(nb):
        pltpu.make_async_copy(out_hbm.at[0, pl.ds(0, _PP), pl.ds(0, _HEAD)],
                              abuf.at[nb], osem[nb]).wait()
        pltpu.make_async_copy(out_hbm.at[0, pl.ds(0, _PP), pl.ds(_HEAD, _TAIL)],
                              cbuf.at[nb], osem[nb]).wait()

    def drain_splat(nb):
        pltpu.make_async_copy(csp_hbm.at[pl.ds(0, _PP)],
                              sbuf.at[nb], ssem).wait()

    prefetch(0, 0)
    prefetch(1, 1)
    do_step(0, 0)
    do_step(1, 1)

    def outer(s2, c):
        for nb in range(2):
            drain_out(nb)
            do_step(s2 * 2 + nb, nb)
        return c
    lax.fori_loop(1, _STEPS // 2, outer, 0)
    drain_out(0)
    drain_out(1)
    drain_splat(0)
    drain_splat(1)


@jax.jit
def _sc_call(flat_idx, cont_splat, disc_table, cont_table):
    mesh = plsc.VectorSubcoreMesh(core_axis_name="c", subcore_axis_name="s")
    f = functools.partial(
        pl.kernel, _sc_body, mesh=mesh,
        out_type=jax.ShapeDtypeStruct((_N // _S, _S, _NROW, _DIM), jnp.float32),
        scratch_types=[
            pltpu.VMEM((_PW * _ND,), jnp.int32),
            pltpu.VMEM((_NCONT, _DIM), jnp.float32),
            pltpu.VMEM((2, _PP, _NCONT, _L), jnp.float32),
            pltpu.VMEM((2, _PP * _ND, _DIM), jnp.float32),
            pltpu.VMEM((2, _PP, _HEAD, _DIM), jnp.float32),
            pltpu.VMEM((2, _PP, _TAIL, _DIM), jnp.float32),
            pltpu.SemaphoreType.DMA,
            pltpu.SemaphoreType.DMA,
            pltpu.SemaphoreType.DMA,
            pltpu.SemaphoreType.DMA,
        ],
    )()
    return f(flat_idx, cont_splat, disc_table, cont_table)


def kernel(discrete_actions, continuous_actions, disc_table, cont_table, offsets):
    b, s, n_disc = discrete_actions.shape
    n_cont = continuous_actions.shape[-1]
    dim = disc_table.shape[-1]
    n = b * s
    flat_idx = (discrete_actions + offsets[None, None, :]).reshape(n * n_disc)
    cont_splat = jnp.broadcast_to(
        continuous_actions.reshape(n, n_cont)[:, :, None], (n, n_cont, _L))
    out = _sc_call(flat_idx, cont_splat, disc_table, cont_table)
    return out.reshape(b, s, n_disc + n_cont, dim)
